# baseline (speedup 1.0000x reference)
import functools

import jax
import jax.numpy as jnp
from jax import lax
from jax.experimental import pallas as pl
from jax.experimental.pallas import tpu as pltpu
from jax.experimental.pallas import tpu_sc as plsc

NC = 2
NS = 16
NW = NC * NS
L = 16
D = 64
DW = D // 4


def _sc_dequant_gather(flat_idx, table_w, scales, B):
  b_per_w = B // NW
  C = 512
  n_chunks = b_per_w // C

  mesh = plsc.VectorSubcoreMesh(
      core_axis_name="c", subcore_axis_name="s", num_cores=NC, num_subcores=NS
  )

  @functools.partial(
      pl.kernel,
      out_type=jax.ShapeDtypeStruct((B, D), jnp.float32),
      mesh=mesh,
      scratch_types=[
          pltpu.VMEM((C,), jnp.int32),
          pltpu.VMEM((C, DW), jnp.int32),
          pltpu.VMEM((C,), jnp.float32),
          pltpu.VMEM((C, D), jnp.float32),
          pltpu.SemaphoreType.DMA,
          pltpu.SemaphoreType.DMA,
      ],
      compiler_params=pltpu.CompilerParams(
          use_tc_tiling_on_sc=False, needs_layout_passes=False),
  )
  def body(idx_hbm, tab_hbm, scl_hbm, out_hbm, idx_v, rows_v, scl_v, out_v,
           sem_r, sem_s):
    wid = lax.axis_index("s") * NC + lax.axis_index("c")
    base = wid * b_per_w
    lanes = jnp.arange(L, dtype=jnp.int32)
    col_idx = [lanes * 4 + k for k in range(4)]

    def chunk_body(c, carry):
      off = base + c * C
      pltpu.sync_copy(idx_hbm.at[pl.ds(off, C)], idx_v)
      cp_r = pltpu.async_copy(tab_hbm.at[idx_v], rows_v, sem_r)
      cp_s = pltpu.async_copy(scl_hbm.at[idx_v], scl_v, sem_s)
      cp_r.wait()
      cp_s.wait()

      def row_body(r, carry2):
        rsplat = jnp.full((L,), r, dtype=jnp.int32)
        w = rows_v[r, :]
        s = plsc.load_gather(scl_v, [rsplat])
        for k in range(4):
          if k < 3:
            v = (w << (24 - 8 * k)) >> 24
          else:
            v = w >> 24
          plsc.store_scatter(out_v, [rsplat, col_idx[k]],
                             v.astype(jnp.float32) * s)
        return carry2

      lax.fori_loop(0, C, row_body, 0)
      pltpu.sync_copy(out_v, out_hbm.at[pl.ds(off, C)])
      return carry

    lax.fori_loop(0, n_chunks, chunk_body, 0)

  return body(flat_idx, table_w, scales)


def kernel(indices, q_table, scales):
  B = indices.size
  flat_idx = indices.reshape(-1).astype(jnp.int32)
  table_w = lax.bitcast_convert_type(
      q_table.reshape(-1, DW, 4), jnp.int32
  )
  out = _sc_dequant_gather(flat_idx, table_w, scales, B)
  return out.reshape(*indices.shape, D)
